# baseline (speedup 1.0000x reference)
import functools

import jax
import jax.numpy as jnp
from jax import lax
from jax.experimental import pallas as pl
from jax.experimental.pallas import tpu as pltpu
from jax.experimental.pallas import tpu_sc as plsc

N = 10000
E = 320000
IN_DIM = 128
HID = 256
OUT_DIM = 10
NG = 64

NC = 2
NS = 16
CH = 128
MB = 16
NPAD = 10240
RPT = NPAD // NS

BLK = 1000
NBLK = N // BLK


def _make_edge_agg(nchunk):
    mesh = plsc.VectorSubcoreMesh(core_axis_name="c", subcore_axis_name="s",
                                  num_cores=NC, num_subcores=NS)

    def body(table, srcs, dsts, ews, zeros, out, src_v, dst_v, ew_v, rows_a,
             rows_b, acc, sem_ga, sem_gb, sem_sa, sem_sb):
        c = lax.axis_index("c")
        w = lax.axis_index("s")

        pltpu.sync_copy(zeros, acc.at[pl.ds(w * RPT, RPT)])
        plsc.subcore_barrier()

        def scale(rows_v, jj):
            for g in range(CH // 16):
                ew16 = ew_v[jj, pl.ds(g * 16, 16)]
                for t in range(16):
                    s = ew16[t]
                    e = g * 16 + t
                    for r in range(8):
                        rows_v[e, pl.ds(r * 16, 16)] = (
                            rows_v[e, pl.ds(r * 16, 16)] * s)

        npair = MB // 2

        def block(b, carry):
            s0 = pl.multiple_of(b * MB, MB)
            pltpu.sync_copy(srcs.at[c, w, pl.ds(s0, MB)], src_v)
            pltpu.sync_copy(dsts.at[c, w, pl.ds(s0, MB)], dst_v)
            pltpu.sync_copy(ews.at[c, w, pl.ds(s0, MB)], ew_v)
            ga0 = pltpu.async_copy(table.at[src_v.at[0]], rows_a, sem_ga)

            def pair(p, carry2):
                ja = 2 * p
                jb = 2 * p + 1
                @pl.when(p > 0)
                def _():
                    pltpu.make_async_copy(rows_b, acc.at[dst_v.at[jb]],
                                          sem_sb).wait()

                pltpu.async_copy(table.at[src_v.at[jb]], rows_b, sem_gb)
                pltpu.make_async_copy(table.at[src_v.at[ja]], rows_a,
                                      sem_ga).wait()
                scale(rows_a, ja)
                sa = pltpu.async_copy(rows_a, acc.at[dst_v.at[ja]], sem_sa,
                                      add=True)
                pltpu.make_async_copy(table.at[src_v.at[jb]], rows_b,
                                      sem_gb).wait()
                scale(rows_b, jb)
                sa.wait()

                @pl.when(p + 1 < npair)
                def _():
                    pltpu.async_copy(table.at[src_v.at[ja + 2]], rows_a,
                                     sem_ga)

                pltpu.async_copy(rows_b, acc.at[dst_v.at[jb]], sem_sb,
                                 add=True)
                return carry2

            lax.fori_loop(0, npair, pair, 0)
            pltpu.make_async_copy(rows_b, acc.at[dst_v.at[MB - 1]],
                                  sem_sb).wait()
            del ga0
            return carry

        lax.fori_loop(0, nchunk // MB, block, 0)
        plsc.subcore_barrier()

        rb = w * RPT
        ob = c * NPAD + rb
        for k in range(RPT // 128):
            pltpu.sync_copy(acc.at[pl.ds(rb + k * 128, 128)],
                            out.at[pl.ds(ob + k * 128, 128)])

    return pl.kernel(
        body,
        out_type=jax.ShapeDtypeStruct((2 * NPAD, 128), jnp.float32),
        mesh=mesh,
        scratch_types=[
            pltpu.VMEM((MB, CH), jnp.int32),
            pltpu.VMEM((MB, CH), jnp.int32),
            pltpu.VMEM((MB, CH), jnp.float32),
            pltpu.VMEM((CH, 128), jnp.float32),
            pltpu.VMEM((CH, 128), jnp.float32),
            pltpu.VMEM_SHARED((NPAD, 128), jnp.float32),
            pltpu.SemaphoreType.DMA,
            pltpu.SemaphoreType.DMA,
            pltpu.SemaphoreType.DMA,
            pltpu.SemaphoreType.DMA,
        ],
    )


def _pad_tile_lists(a, per_tile, pad_to, lead):
    a = a.reshape(lead + (per_tile,))
    a = jnp.pad(a, [(0, 0)] * len(lead) + [(0, pad_to - per_tile)])
    return a.reshape(lead + (pad_to // CH, CH))


def _y_body(alo, ahi, hlo, hhi, wr, br, wo, y_ref, st_ref):
    i = pl.program_id(0)
    dh = alo.shape[1]
    y = (jnp.dot(alo[...], wr[0:dh, :], preferred_element_type=jnp.float32)
         + jnp.dot(ahi[...], wr[dh:2 * dh, :], preferred_element_type=jnp.float32)
         + jnp.dot(hlo[...], wo[0:dh, :], preferred_element_type=jnp.float32)
         + jnp.dot(hhi[...], wo[dh:2 * dh, :], preferred_element_type=jnp.float32)
         + br[...])
    y_ref[...] = y

    @pl.when(i == 0)
    def _():
        st_ref[...] = jnp.zeros_like(st_ref)

    st_ref[0:1, :] += jnp.sum(y, axis=0, keepdims=True)
    st_ref[1:2, :] += jnp.sum(y * y, axis=0, keepdims=True)


def _y_call(agg, hs, W_rel, b_rel, W_root):
    dh = agg.shape[1]
    alo, ahi = agg[:N], agg[NPAD:NPAD + N]
    lo = pl.BlockSpec((BLK, dh), lambda i: (i, 0))
    hi = lo
    full = lambda a: pl.BlockSpec(a.shape, lambda i: (0,) * a.ndim)
    return pl.pallas_call(
        _y_body,
        grid=(NBLK,),
        in_specs=[lo, hi, lo, hi, full(W_rel), full(b_rel), full(W_root)],
        out_specs=[pl.BlockSpec((BLK, HID), lambda i: (i, 0)),
                   pl.BlockSpec((8, HID), lambda i: (0, 0))],
        out_shape=[jax.ShapeDtypeStruct((N, HID), jnp.float32),
                   jax.ShapeDtypeStruct((8, HID), jnp.float32)],
    )(alo, ahi, hs[:N], hs[N:], W_rel, b_rel, W_root)


def _bn_body(y_ref, st_ref, g_ref, b_ref, o_ref):
    mu = st_ref[0:1, :] / N
    var = st_ref[1:2, :] / N - mu * mu
    inv = lax.rsqrt(var + 1e-5)
    h = jnp.maximum((y_ref[...] - mu) * inv * g_ref[...] + b_ref[...], 0.0)
    o_ref[0, :, :] = h[:, 0:HID // 2]
    o_ref[1, :, :] = h[:, HID // 2:HID]


def _bn_call(y, st, gamma, beta):
    full = lambda a: pl.BlockSpec(a.shape, lambda i: (0,) * a.ndim)
    return pl.pallas_call(
        _bn_body,
        grid=(NBLK,),
        in_specs=[pl.BlockSpec((BLK, HID), lambda i: (i, 0)), full(st),
                  full(gamma), full(beta)],
        out_specs=pl.BlockSpec((2, BLK, HID // 2), lambda i: (0, i, 0)),
        out_shape=jax.ShapeDtypeStruct((2, N, HID // 2), jnp.float32),
    )(y, st, gamma, beta)


def _pool_body(alo, ahi, hlo, hhi, wr, br, wo, b_ref, sum_ref, max_ref):
    i = pl.program_id(0)
    dh = alo.shape[1]
    h3 = (jnp.dot(alo[...], wr[0:dh, :], preferred_element_type=jnp.float32)
          + jnp.dot(ahi[...], wr[dh:2 * dh, :], preferred_element_type=jnp.float32)
          + jnp.dot(hlo[...], wo[0:dh, :], preferred_element_type=jnp.float32)
          + jnp.dot(hhi[...], wo[dh:2 * dh, :], preferred_element_type=jnp.float32)
          + br[...])
    ids = b_ref[...]

    @pl.when(i == 0)
    def _():
        sum_ref[...] = jnp.zeros_like(sum_ref)
        max_ref[...] = jnp.full_like(max_ref, -jnp.inf)

    onehot = (ids == lax.broadcasted_iota(jnp.int32, (1, NG), 1)
              ).astype(jnp.float32)
    sum_ref[...] += lax.dot_general(onehot, h3, (((0,), (0,)), ((), ())),
                                    preferred_element_type=jnp.float32)

    lo_g = jnp.min(ids)
    hi_g = jnp.max(ids)

    def gbody(g, carry):
        m = jnp.max(jnp.where(ids == g, h3, -jnp.inf), axis=0,
                    keepdims=True)
        max_ref[pl.ds(g, 1), :] = jnp.maximum(max_ref[pl.ds(g, 1), :], m)
        return carry

    lax.fori_loop(lo_g, hi_g + 1, gbody, 0)


def _pool_call(agg, hs, W_rel, b_rel, W_root, batch3):
    dh = agg.shape[1]
    alo, ahi = agg[:N], agg[NPAD:NPAD + N]
    lo = pl.BlockSpec((BLK, dh), lambda i: (i, 0))
    hi = lo
    full = lambda a: pl.BlockSpec(a.shape, lambda i: (0,) * a.ndim)
    return pl.pallas_call(
        _pool_body,
        grid=(NBLK,),
        in_specs=[lo, hi, lo, hi, full(W_rel), full(b_rel), full(W_root),
                  pl.BlockSpec((BLK, 1), lambda i: (i, 0))],
        out_specs=[pl.BlockSpec((NG, HID), lambda i: (0, 0)),
                   pl.BlockSpec((NG, HID), lambda i: (0, 0))],
        out_shape=[jax.ShapeDtypeStruct((NG, HID), jnp.float32),
                   jax.ShapeDtypeStruct((NG, HID), jnp.float32)],
    )(alo, ahi, hs[:N], hs[N:], W_rel, b_rel, W_root, batch3)


def _head_body(sum_ref, max_ref, b_ref, w1, b1, w2, b2, o_ref):
    ids = b_ref[...]
    onehot = (ids == lax.broadcasted_iota(jnp.int32, (1, NG), 1)
              ).astype(jnp.float32)
    ones = jnp.ones((N, 1), jnp.float32)
    cnt = jnp.maximum(lax.dot_general(onehot, ones, (((0,), (0,)), ((), ())),
                                      preferred_element_type=jnp.float32), 1.0)
    s = sum_ref[...]
    mean = s / cnt
    mx = max_ref[...]
    a = (jnp.dot(s, w1[0:HID, :], preferred_element_type=jnp.float32)
         + jnp.dot(mean, w1[HID:2 * HID, :], preferred_element_type=jnp.float32)
         + jnp.dot(mx, w1[2 * HID:3 * HID, :], preferred_element_type=jnp.float32)
         + b1[...])
    a = jnp.maximum(a, 0.0)
    o_ref[...] = jnp.dot(a, w2[...], preferred_element_type=jnp.float32) + b2[...]


def _head_call(sumpool, maxpool, batch2, W1, b1, W2p, b2p):
    return pl.pallas_call(
        _head_body,
        out_shape=jax.ShapeDtypeStruct((NG, 128), jnp.float32),
    )(sumpool, maxpool, batch2, W1, b1, W2p, b2p)


def kernel(x, edge_index, edge_weight, batch, W_rel0, b_rel0, W_root0,
           W_rel1, b_rel1, W_root1, W_rel2, b_rel2, W_root2,
           gamma0, beta0, gamma1, beta1, W1, b1, W2, b2):
    order = jnp.argsort(edge_index[0])
    src = edge_index[0][order]
    dst = edge_index[1][order]
    edge_weight = edge_weight[order]
    ept0 = E // (NC * NS)
    pad0 = MB * CH * -(-ept0 // (MB * CH))
    src0 = _pad_tile_lists(src, ept0, pad0, (NC, NS))
    src0 = src0 + (jnp.arange(NC, dtype=jnp.int32) * N)[:, None, None, None]
    dst0 = _pad_tile_lists(dst, ept0, pad0, (NC, NS))
    ew0 = _pad_tile_lists(edge_weight, ept0, pad0, (NC, NS))
    ept12 = E // NS
    pad12 = MB * CH * -(-ept12 // (MB * CH))
    srcr = _pad_tile_lists(src, ept12, pad12, (NS,))
    src12 = jnp.stack([srcr, srcr + N])
    dstr = _pad_tile_lists(dst, ept12, pad12, (NS,))
    dst12 = jnp.stack([dstr, dstr])
    ewr = _pad_tile_lists(edge_weight, ept12, pad12, (NS,))
    ew12 = jnp.stack([ewr, ewr])
    zeros = jnp.zeros((RPT, 128), jnp.float32)

    agg_half = _make_edge_agg(pad0 // CH)
    agg_full = _make_edge_agg(pad12 // CH)

    x2 = jnp.concatenate([x, x], axis=0)
    agg0 = agg_half(x2, src0, dst0, ew0, zeros)
    wr0 = jnp.concatenate([W_rel0, W_rel0], axis=0)
    wo0 = jnp.concatenate([W_root0, jnp.zeros_like(W_root0)], axis=0)
    xs = jnp.concatenate([x, x], axis=0)
    y0, st0 = _y_call(agg0, xs, wr0, b_rel0.reshape(1, HID), wo0)
    h1 = _bn_call(y0, st0, gamma0.reshape(1, HID),
                  beta0.reshape(1, HID)).reshape(2 * N, HID // 2)

    agg1 = agg_full(h1, src12, dst12, ew12, zeros)
    y1, st1 = _y_call(agg1, h1, W_rel1, b_rel1.reshape(1, HID), W_root1)
    h2 = _bn_call(y1, st1, gamma1.reshape(1, HID),
                  beta1.reshape(1, HID)).reshape(2 * N, HID // 2)

    agg2 = agg_full(h2, src12, dst12, ew12, zeros)
    sumpool, maxpool = _pool_call(agg2, h2, W_rel2, b_rel2.reshape(1, HID),
                                  W_root2, batch.reshape(N, 1))

    W2p = jnp.pad(W2, ((0, 0), (0, 128 - OUT_DIM)))
    b2p = jnp.pad(b2, (0, 128 - OUT_DIM)).reshape(1, 128)
    out = _head_call(sumpool, maxpool, batch.reshape(N, 1), W1,
                     b1.reshape(1, HID), W2p, b2p)
    return out[:, :OUT_DIM]


# speedup vs baseline: 1.4339x; 1.4339x over previous
import functools

import jax
import jax.numpy as jnp
from jax import lax
from jax.experimental import pallas as pl
from jax.experimental.pallas import tpu as pltpu
from jax.experimental.pallas import tpu_sc as plsc

N = 10000
E = 320000
IN_DIM = 128
HID = 256
OUT_DIM = 10
NG = 64

NC = 2
NS = 16
CH = 128
MB = 16
NPAD = 10240
RPT = NPAD // NS

BLK = 1000
NBLK = N // BLK


def _make_edge_agg(nchunk):
    mesh = plsc.VectorSubcoreMesh(core_axis_name="c", subcore_axis_name="s",
                                  num_cores=NC, num_subcores=NS)

    def body(table, srcs, dsts, ews, zeros, out, src_v, dst_v, ew_v, rows_a,
             rows_b, acc, sem_ga, sem_ga2, sem_gb, sem_gb2, sem_sa, sem_sb):
        c = lax.axis_index("c")
        w = lax.axis_index("s")

        pltpu.sync_copy(zeros, acc.at[pl.ds(w * RPT, RPT)])
        plsc.subcore_barrier()

        H = CH // 2

        def gather2(jj, rows_v, s1, s2):
            pltpu.async_copy(table.at[src_v.at[jj, pl.ds(0, H)]],
                             rows_v.at[pl.ds(0, H)], s1)
            pltpu.async_copy(table.at[src_v.at[jj, pl.ds(H, H)]],
                             rows_v.at[pl.ds(H, H)], s2)

        def gwait2(jj, rows_v, s1, s2):
            pltpu.make_async_copy(table.at[src_v.at[jj, pl.ds(0, H)]],
                                  rows_v.at[pl.ds(0, H)], s1).wait()
            pltpu.make_async_copy(table.at[src_v.at[jj, pl.ds(H, H)]],
                                  rows_v.at[pl.ds(H, H)], s2).wait()

        def scale(rows_v, jj):
            for g in range(CH // 16):
                ew16 = ew_v[jj, pl.ds(g * 16, 16)]
                for t in range(16):
                    s = ew16[t]
                    e = g * 16 + t
                    for r in range(8):
                        rows_v[e, pl.ds(r * 16, 16)] = (
                            rows_v[e, pl.ds(r * 16, 16)] * s)

        npair = MB // 2

        def block(b, carry):
            s0 = pl.multiple_of(b * MB, MB)
            pltpu.sync_copy(srcs.at[c, w, pl.ds(s0, MB)], src_v)
            pltpu.sync_copy(dsts.at[c, w, pl.ds(s0, MB)], dst_v)
            pltpu.sync_copy(ews.at[c, w, pl.ds(s0, MB)], ew_v)
            gather2(0, rows_a, sem_ga, sem_ga2)

            def pair(p, carry2):
                ja = 2 * p
                jb = 2 * p + 1
                @pl.when(p > 0)
                def _():
                    pltpu.make_async_copy(rows_b, acc.at[dst_v.at[jb]],
                                          sem_sb).wait()

                gather2(jb, rows_b, sem_gb, sem_gb2)
                gwait2(ja, rows_a, sem_ga, sem_ga2)
                scale(rows_a, ja)
                sa = pltpu.async_copy(rows_a, acc.at[dst_v.at[ja]], sem_sa,
                                      add=True)
                gwait2(jb, rows_b, sem_gb, sem_gb2)
                scale(rows_b, jb)
                sa.wait()

                @pl.when(p + 1 < npair)
                def _():
                    gather2(ja + 2, rows_a, sem_ga, sem_ga2)

                pltpu.async_copy(rows_b, acc.at[dst_v.at[jb]], sem_sb,
                                 add=True)
                return carry2

            lax.fori_loop(0, npair, pair, 0)
            pltpu.make_async_copy(rows_b, acc.at[dst_v.at[MB - 1]],
                                  sem_sb).wait()
            return carry

        lax.fori_loop(0, nchunk // MB, block, 0)
        plsc.subcore_barrier()

        rb = w * RPT
        ob = c * NPAD + rb
        for k in range(RPT // 128):
            pltpu.sync_copy(acc.at[pl.ds(rb + k * 128, 128)],
                            out.at[pl.ds(ob + k * 128, 128)])

    return pl.kernel(
        body,
        out_type=jax.ShapeDtypeStruct((2 * NPAD, 128), jnp.float32),
        mesh=mesh,
        scratch_types=[
            pltpu.VMEM((MB, CH), jnp.int32),
            pltpu.VMEM((MB, CH), jnp.int32),
            pltpu.VMEM((MB, CH), jnp.float32),
            pltpu.VMEM((CH, 128), jnp.float32),
            pltpu.VMEM((CH, 128), jnp.float32),
            pltpu.VMEM_SHARED((NPAD, 128), jnp.float32),
            pltpu.SemaphoreType.DMA,
            pltpu.SemaphoreType.DMA,
            pltpu.SemaphoreType.DMA,
            pltpu.SemaphoreType.DMA,
            pltpu.SemaphoreType.DMA,
            pltpu.SemaphoreType.DMA,
        ],
    )


def _pad_tile_lists(a, per_tile, pad_to, lead):
    a = a.reshape(lead + (per_tile,))
    a = jnp.pad(a, [(0, 0)] * len(lead) + [(0, pad_to - per_tile)])
    return a.reshape(lead + (pad_to // CH, CH))


def _y_body(alo, ahi, hlo, hhi, wr, br, wo, y_ref, st_ref):
    i = pl.program_id(0)
    dh = alo.shape[1]
    y = (jnp.dot(alo[...], wr[0:dh, :], preferred_element_type=jnp.float32)
         + jnp.dot(ahi[...], wr[dh:2 * dh, :], preferred_element_type=jnp.float32)
         + jnp.dot(hlo[...], wo[0:dh, :], preferred_element_type=jnp.float32)
         + jnp.dot(hhi[...], wo[dh:2 * dh, :], preferred_element_type=jnp.float32)
         + br[...])
    y_ref[...] = y

    @pl.when(i == 0)
    def _():
        st_ref[...] = jnp.zeros_like(st_ref)

    st_ref[0:1, :] += jnp.sum(y, axis=0, keepdims=True)
    st_ref[1:2, :] += jnp.sum(y * y, axis=0, keepdims=True)


def _y_call(agg, hs, W_rel, b_rel, W_root):
    dh = agg.shape[1]
    alo, ahi = agg[:N], agg[NPAD:NPAD + N]
    lo = pl.BlockSpec((BLK, dh), lambda i: (i, 0))
    hi = lo
    full = lambda a: pl.BlockSpec(a.shape, lambda i: (0,) * a.ndim)
    return pl.pallas_call(
        _y_body,
        grid=(NBLK,),
        in_specs=[lo, hi, lo, hi, full(W_rel), full(b_rel), full(W_root)],
        out_specs=[pl.BlockSpec((BLK, HID), lambda i: (i, 0)),
                   pl.BlockSpec((8, HID), lambda i: (0, 0))],
        out_shape=[jax.ShapeDtypeStruct((N, HID), jnp.float32),
                   jax.ShapeDtypeStruct((8, HID), jnp.float32)],
    )(alo, ahi, hs[:N], hs[N:], W_rel, b_rel, W_root)


def _bn_body(y_ref, st_ref, g_ref, b_ref, o_ref):
    mu = st_ref[0:1, :] / N
    var = st_ref[1:2, :] / N - mu * mu
    inv = lax.rsqrt(var + 1e-5)
    h = jnp.maximum((y_ref[...] - mu) * inv * g_ref[...] + b_ref[...], 0.0)
    o_ref[0, :, :] = h[:, 0:HID // 2]
    o_ref[1, :, :] = h[:, HID // 2:HID]


def _bn_call(y, st, gamma, beta):
    full = lambda a: pl.BlockSpec(a.shape, lambda i: (0,) * a.ndim)
    return pl.pallas_call(
        _bn_body,
        grid=(NBLK,),
        in_specs=[pl.BlockSpec((BLK, HID), lambda i: (i, 0)), full(st),
                  full(gamma), full(beta)],
        out_specs=pl.BlockSpec((2, BLK, HID // 2), lambda i: (0, i, 0)),
        out_shape=jax.ShapeDtypeStruct((2, N, HID // 2), jnp.float32),
    )(y, st, gamma, beta)


def _pool_body(alo, ahi, hlo, hhi, wr, br, wo, b_ref, sum_ref, max_ref):
    i = pl.program_id(0)
    dh = alo.shape[1]
    h3 = (jnp.dot(alo[...], wr[0:dh, :], preferred_element_type=jnp.float32)
          + jnp.dot(ahi[...], wr[dh:2 * dh, :], preferred_element_type=jnp.float32)
          + jnp.dot(hlo[...], wo[0:dh, :], preferred_element_type=jnp.float32)
          + jnp.dot(hhi[...], wo[dh:2 * dh, :], preferred_element_type=jnp.float32)
          + br[...])
    ids = b_ref[...]

    @pl.when(i == 0)
    def _():
        sum_ref[...] = jnp.zeros_like(sum_ref)
        max_ref[...] = jnp.full_like(max_ref, -jnp.inf)

    onehot = (ids == lax.broadcasted_iota(jnp.int32, (1, NG), 1)
              ).astype(jnp.float32)
    sum_ref[...] += lax.dot_general(onehot, h3, (((0,), (0,)), ((), ())),
                                    preferred_element_type=jnp.float32)

    lo_g = jnp.min(ids)
    hi_g = jnp.max(ids)

    def gbody(g, carry):
        m = jnp.max(jnp.where(ids == g, h3, -jnp.inf), axis=0,
                    keepdims=True)
        max_ref[pl.ds(g, 1), :] = jnp.maximum(max_ref[pl.ds(g, 1), :], m)
        return carry

    lax.fori_loop(lo_g, hi_g + 1, gbody, 0)


def _pool_call(agg, hs, W_rel, b_rel, W_root, batch3):
    dh = agg.shape[1]
    alo, ahi = agg[:N], agg[NPAD:NPAD + N]
    lo = pl.BlockSpec((BLK, dh), lambda i: (i, 0))
    hi = lo
    full = lambda a: pl.BlockSpec(a.shape, lambda i: (0,) * a.ndim)
    return pl.pallas_call(
        _pool_body,
        grid=(NBLK,),
        in_specs=[lo, hi, lo, hi, full(W_rel), full(b_rel), full(W_root),
                  pl.BlockSpec((BLK, 1), lambda i: (i, 0))],
        out_specs=[pl.BlockSpec((NG, HID), lambda i: (0, 0)),
                   pl.BlockSpec((NG, HID), lambda i: (0, 0))],
        out_shape=[jax.ShapeDtypeStruct((NG, HID), jnp.float32),
                   jax.ShapeDtypeStruct((NG, HID), jnp.float32)],
    )(alo, ahi, hs[:N], hs[N:], W_rel, b_rel, W_root, batch3)


def _head_body(sum_ref, max_ref, b_ref, w1, b1, w2, b2, o_ref):
    ids = b_ref[...]
    onehot = (ids == lax.broadcasted_iota(jnp.int32, (1, NG), 1)
              ).astype(jnp.float32)
    ones = jnp.ones((N, 1), jnp.float32)
    cnt = jnp.maximum(lax.dot_general(onehot, ones, (((0,), (0,)), ((), ())),
                                      preferred_element_type=jnp.float32), 1.0)
    s = sum_ref[...]
    mean = s / cnt
    mx = max_ref[...]
    a = (jnp.dot(s, w1[0:HID, :], preferred_element_type=jnp.float32)
         + jnp.dot(mean, w1[HID:2 * HID, :], preferred_element_type=jnp.float32)
         + jnp.dot(mx, w1[2 * HID:3 * HID, :], preferred_element_type=jnp.float32)
         + b1[...])
    a = jnp.maximum(a, 0.0)
    o_ref[...] = jnp.dot(a, w2[...], preferred_element_type=jnp.float32) + b2[...]


def _head_call(sumpool, maxpool, batch2, W1, b1, W2p, b2p):
    return pl.pallas_call(
        _head_body,
        out_shape=jax.ShapeDtypeStruct((NG, 128), jnp.float32),
    )(sumpool, maxpool, batch2, W1, b1, W2p, b2p)


def kernel(x, edge_index, edge_weight, batch, W_rel0, b_rel0, W_root0,
           W_rel1, b_rel1, W_root1, W_rel2, b_rel2, W_root2,
           gamma0, beta0, gamma1, beta1, W1, b1, W2, b2):
    src, dst = edge_index[0], edge_index[1]
    ept0 = E // (NC * NS)
    pad0 = MB * CH * -(-ept0 // (MB * CH))
    src0 = _pad_tile_lists(src, ept0, pad0, (NC, NS))
    src0 = src0 + (jnp.arange(NC, dtype=jnp.int32) * N)[:, None, None, None]
    dst0 = _pad_tile_lists(dst, ept0, pad0, (NC, NS))
    ew0 = _pad_tile_lists(edge_weight, ept0, pad0, (NC, NS))
    ept12 = E // NS
    pad12 = MB * CH * -(-ept12 // (MB * CH))
    srcr = _pad_tile_lists(src, ept12, pad12, (NS,))
    src12 = jnp.stack([srcr, srcr + N])
    dstr = _pad_tile_lists(dst, ept12, pad12, (NS,))
    dst12 = jnp.stack([dstr, dstr])
    ewr = _pad_tile_lists(edge_weight, ept12, pad12, (NS,))
    ew12 = jnp.stack([ewr, ewr])
    zeros = jnp.zeros((RPT, 128), jnp.float32)

    agg_half = _make_edge_agg(pad0 // CH)
    agg_full = _make_edge_agg(pad12 // CH)

    x2 = jnp.concatenate([x, x], axis=0)
    agg0 = agg_half(x2, src0, dst0, ew0, zeros)
    wr0 = jnp.concatenate([W_rel0, W_rel0], axis=0)
    wo0 = jnp.concatenate([W_root0, jnp.zeros_like(W_root0)], axis=0)
    xs = jnp.concatenate([x, x], axis=0)
    y0, st0 = _y_call(agg0, xs, wr0, b_rel0.reshape(1, HID), wo0)
    h1 = _bn_call(y0, st0, gamma0.reshape(1, HID),
                  beta0.reshape(1, HID)).reshape(2 * N, HID // 2)

    agg1 = agg_full(h1, src12, dst12, ew12, zeros)
    y1, st1 = _y_call(agg1, h1, W_rel1, b_rel1.reshape(1, HID), W_root1)
    h2 = _bn_call(y1, st1, gamma1.reshape(1, HID),
                  beta1.reshape(1, HID)).reshape(2 * N, HID // 2)

    agg2 = agg_full(h2, src12, dst12, ew12, zeros)
    sumpool, maxpool = _pool_call(agg2, h2, W_rel2, b_rel2.reshape(1, HID),
                                  W_root2, batch.reshape(N, 1))

    W2p = jnp.pad(W2, ((0, 0), (0, 128 - OUT_DIM)))
    b2p = jnp.pad(b2, (0, 128 - OUT_DIM)).reshape(1, 128)
    out = _head_call(sumpool, maxpool, batch.reshape(N, 1), W1,
                     b1.reshape(1, HID), W2p, b2p)
    return out[:, :OUT_DIM]


# speedup vs baseline: 1.4432x; 1.0065x over previous
import functools

import jax
import jax.numpy as jnp
from jax import lax
from jax.experimental import pallas as pl
from jax.experimental.pallas import tpu as pltpu
from jax.experimental.pallas import tpu_sc as plsc

N = 10000
E = 320000
IN_DIM = 128
HID = 256
OUT_DIM = 10
NG = 64

NC = 2
NS = 16
CH = 128
MB = 16
NPAD = 10240
RPT = NPAD // NS

BLK = 1000
NBLK = N // BLK


def _make_edge_agg(nchunk):
    mesh = plsc.VectorSubcoreMesh(core_axis_name="c", subcore_axis_name="s",
                                  num_cores=NC, num_subcores=NS)

    def body(table, srcs, dsts, ews, zeros, out, src_v, dst_v, ew_v, rows_a,
             rows_b, acc, sem_ga, sem_gb, sem_sa, sem_sb):
        c = lax.axis_index("c")
        w = lax.axis_index("s")

        pltpu.sync_copy(zeros, acc.at[pl.ds(w * RPT, RPT)])
        plsc.subcore_barrier()

        def scale(rows_v, jj):
            for g in range(CH // 16):
                ew16 = ew_v[jj, pl.ds(g * 16, 16)]
                for t in range(16):
                    s = ew16[t]
                    e = g * 16 + t
                    for r in range(8):
                        rows_v[e, pl.ds(r * 16, 16)] = (
                            rows_v[e, pl.ds(r * 16, 16)] * s)

        npair = MB // 2

        def block(b, carry):
            s0 = pl.multiple_of(b * MB, MB)
            pltpu.sync_copy(srcs.at[c, w, pl.ds(s0, MB)], src_v)
            pltpu.sync_copy(dsts.at[c, w, pl.ds(s0, MB)], dst_v)
            pltpu.sync_copy(ews.at[c, w, pl.ds(s0, MB)], ew_v)
            ga0 = pltpu.async_copy(table.at[src_v.at[0]], rows_a, sem_ga)

            def pair(p, carry2):
                ja = 2 * p
                jb = 2 * p + 1
                @pl.when(p > 0)
                def _():
                    pltpu.make_async_copy(rows_b, acc.at[dst_v.at[jb]],
                                          sem_sb).wait()

                pltpu.async_copy(table.at[src_v.at[jb]], rows_b, sem_gb)
                pltpu.make_async_copy(table.at[src_v.at[ja]], rows_a,
                                      sem_ga).wait()
                scale(rows_a, ja)
                sa = pltpu.async_copy(rows_a, acc.at[dst_v.at[ja]], sem_sa,
                                      add=True)
                pltpu.make_async_copy(table.at[src_v.at[jb]], rows_b,
                                      sem_gb).wait()
                scale(rows_b, jb)
                sa.wait()

                @pl.when(p + 1 < npair)
                def _():
                    pltpu.async_copy(table.at[src_v.at[ja + 2]], rows_a,
                                     sem_ga)

                pltpu.async_copy(rows_b, acc.at[dst_v.at[jb]], sem_sb,
                                 add=True)
                return carry2

            lax.fori_loop(0, npair, pair, 0)
            pltpu.make_async_copy(rows_b, acc.at[dst_v.at[MB - 1]],
                                  sem_sb).wait()
            del ga0
            return carry

        lax.fori_loop(0, nchunk // MB, block, 0)
        plsc.subcore_barrier()

        rb = w * RPT
        ob = c * NPAD + rb
        for k in range(RPT // 128):
            pltpu.sync_copy(acc.at[pl.ds(rb + k * 128, 128)],
                            out.at[pl.ds(ob + k * 128, 128)])

    return pl.kernel(
        body,
        out_type=jax.ShapeDtypeStruct((2 * NPAD, 128), jnp.float32),
        mesh=mesh,
        scratch_types=[
            pltpu.VMEM((MB, CH), jnp.int32),
            pltpu.VMEM((MB, CH), jnp.int32),
            pltpu.VMEM((MB, CH), jnp.float32),
            pltpu.VMEM((CH, 128), jnp.float32),
            pltpu.VMEM((CH, 128), jnp.float32),
            pltpu.VMEM_SHARED((NPAD, 128), jnp.float32),
            pltpu.SemaphoreType.DMA,
            pltpu.SemaphoreType.DMA,
            pltpu.SemaphoreType.DMA,
            pltpu.SemaphoreType.DMA,
        ],
    )


def _pad_tile_lists(a, per_tile, pad_to, lead):
    a = a.reshape(lead + (per_tile,))
    a = jnp.pad(a, [(0, 0)] * len(lead) + [(0, pad_to - per_tile)])
    return a.reshape(lead + (pad_to // CH, CH))


def _y_body(alo, ahi, hlo, hhi, wr, br, wo, y_ref, st_ref):
    i = pl.program_id(0)
    dh = alo.shape[1]
    y = (jnp.dot(alo[...], wr[0:dh, :], preferred_element_type=jnp.float32)
         + jnp.dot(ahi[...], wr[dh:2 * dh, :], preferred_element_type=jnp.float32)
         + jnp.dot(hlo[...], wo[0:dh, :], preferred_element_type=jnp.float32)
         + jnp.dot(hhi[...], wo[dh:2 * dh, :], preferred_element_type=jnp.float32)
         + br[...])
    y_ref[...] = y

    @pl.when(i == 0)
    def _():
        st_ref[...] = jnp.zeros_like(st_ref)

    st_ref[0:1, :] += jnp.sum(y, axis=0, keepdims=True)
    st_ref[1:2, :] += jnp.sum(y * y, axis=0, keepdims=True)


def _y_call(agg, hs, W_rel, b_rel, W_root):
    dh = agg.shape[1]
    alo, ahi = agg[:N], agg[NPAD:NPAD + N]
    lo = pl.BlockSpec((BLK, dh), lambda i: (i, 0))
    hi = lo
    full = lambda a: pl.BlockSpec(a.shape, lambda i: (0,) * a.ndim)
    return pl.pallas_call(
        _y_body,
        grid=(NBLK,),
        in_specs=[lo, hi, lo, hi, full(W_rel), full(b_rel), full(W_root)],
        out_specs=[pl.BlockSpec((BLK, HID), lambda i: (i, 0)),
                   pl.BlockSpec((8, HID), lambda i: (0, 0))],
        out_shape=[jax.ShapeDtypeStruct((N, HID), jnp.float32),
                   jax.ShapeDtypeStruct((8, HID), jnp.float32)],
    )(alo, ahi, hs[:N], hs[N:], W_rel, b_rel, W_root)


def _bn_body(y_ref, st_ref, g_ref, b_ref, o_ref):
    mu = st_ref[0:1, :] / N
    var = st_ref[1:2, :] / N - mu * mu
    inv = lax.rsqrt(var + 1e-5)
    h = jnp.maximum((y_ref[...] - mu) * inv * g_ref[...] + b_ref[...], 0.0)
    o_ref[0, :, :] = h[:, 0:HID // 2]
    o_ref[1, :, :] = h[:, HID // 2:HID]


def _bn_call(y, st, gamma, beta):
    full = lambda a: pl.BlockSpec(a.shape, lambda i: (0,) * a.ndim)
    return pl.pallas_call(
        _bn_body,
        grid=(NBLK,),
        in_specs=[pl.BlockSpec((BLK, HID), lambda i: (i, 0)), full(st),
                  full(gamma), full(beta)],
        out_specs=pl.BlockSpec((2, BLK, HID // 2), lambda i: (0, i, 0)),
        out_shape=jax.ShapeDtypeStruct((2, N, HID // 2), jnp.float32),
    )(y, st, gamma, beta)


def _pool_body(alo, ahi, hlo, hhi, wr, br, wo, b_ref, sum_ref, max_ref):
    i = pl.program_id(0)
    dh = alo.shape[1]
    h3 = (jnp.dot(alo[...], wr[0:dh, :], preferred_element_type=jnp.float32)
          + jnp.dot(ahi[...], wr[dh:2 * dh, :], preferred_element_type=jnp.float32)
          + jnp.dot(hlo[...], wo[0:dh, :], preferred_element_type=jnp.float32)
          + jnp.dot(hhi[...], wo[dh:2 * dh, :], preferred_element_type=jnp.float32)
          + br[...])
    ids = b_ref[...]

    @pl.when(i == 0)
    def _():
        sum_ref[...] = jnp.zeros_like(sum_ref)
        max_ref[...] = jnp.full_like(max_ref, -jnp.inf)

    onehot = (ids == lax.broadcasted_iota(jnp.int32, (1, NG), 1)
              ).astype(jnp.float32)
    sum_ref[...] += lax.dot_general(onehot, h3, (((0,), (0,)), ((), ())),
                                    preferred_element_type=jnp.float32)

    lo_g = jnp.min(ids)
    hi_g = jnp.max(ids)

    def gbody(g, carry):
        m = jnp.max(jnp.where(ids == g, h3, -jnp.inf), axis=0,
                    keepdims=True)
        max_ref[pl.ds(g, 1), :] = jnp.maximum(max_ref[pl.ds(g, 1), :], m)
        return carry

    lax.fori_loop(lo_g, hi_g + 1, gbody, 0)


def _pool_call(agg, hs, W_rel, b_rel, W_root, batch3):
    dh = agg.shape[1]
    alo, ahi = agg[:N], agg[NPAD:NPAD + N]
    lo = pl.BlockSpec((BLK, dh), lambda i: (i, 0))
    hi = lo
    full = lambda a: pl.BlockSpec(a.shape, lambda i: (0,) * a.ndim)
    return pl.pallas_call(
        _pool_body,
        grid=(NBLK,),
        in_specs=[lo, hi, lo, hi, full(W_rel), full(b_rel), full(W_root),
                  pl.BlockSpec((BLK, 1), lambda i: (i, 0))],
        out_specs=[pl.BlockSpec((NG, HID), lambda i: (0, 0)),
                   pl.BlockSpec((NG, HID), lambda i: (0, 0))],
        out_shape=[jax.ShapeDtypeStruct((NG, HID), jnp.float32),
                   jax.ShapeDtypeStruct((NG, HID), jnp.float32)],
    )(alo, ahi, hs[:N], hs[N:], W_rel, b_rel, W_root, batch3)


def _head_body(sum_ref, max_ref, b_ref, w1, b1, w2, b2, o_ref):
    ids = b_ref[...]
    onehot = (ids == lax.broadcasted_iota(jnp.int32, (1, NG), 1)
              ).astype(jnp.float32)
    ones = jnp.ones((N, 1), jnp.float32)
    cnt = jnp.maximum(lax.dot_general(onehot, ones, (((0,), (0,)), ((), ())),
                                      preferred_element_type=jnp.float32), 1.0)
    s = sum_ref[...]
    mean = s / cnt
    mx = max_ref[...]
    a = (jnp.dot(s, w1[0:HID, :], preferred_element_type=jnp.float32)
         + jnp.dot(mean, w1[HID:2 * HID, :], preferred_element_type=jnp.float32)
         + jnp.dot(mx, w1[2 * HID:3 * HID, :], preferred_element_type=jnp.float32)
         + b1[...])
    a = jnp.maximum(a, 0.0)
    o_ref[...] = jnp.dot(a, w2[...], preferred_element_type=jnp.float32) + b2[...]


def _head_call(sumpool, maxpool, batch2, W1, b1, W2p, b2p):
    return pl.pallas_call(
        _head_body,
        out_shape=jax.ShapeDtypeStruct((NG, 128), jnp.float32),
    )(sumpool, maxpool, batch2, W1, b1, W2p, b2p)


def kernel(x, edge_index, edge_weight, batch, W_rel0, b_rel0, W_root0,
           W_rel1, b_rel1, W_root1, W_rel2, b_rel2, W_root2,
           gamma0, beta0, gamma1, beta1, W1, b1, W2, b2):
    src, dst = edge_index[0], edge_index[1]
    ept0 = E // (NC * NS)
    pad0 = MB * CH * -(-ept0 // (MB * CH))
    src0 = _pad_tile_lists(src, ept0, pad0, (NC, NS))
    src0 = src0 + (jnp.arange(NC, dtype=jnp.int32) * N)[:, None, None, None]
    dst0 = _pad_tile_lists(dst, ept0, pad0, (NC, NS))
    ew0 = _pad_tile_lists(edge_weight, ept0, pad0, (NC, NS))
    ept12 = E // NS
    pad12 = MB * CH * -(-ept12 // (MB * CH))
    srcr = _pad_tile_lists(src, ept12, pad12, (NS,))
    src12 = jnp.stack([srcr, srcr + N])
    dstr = _pad_tile_lists(dst, ept12, pad12, (NS,))
    dst12 = jnp.stack([dstr, dstr])
    ewr = _pad_tile_lists(edge_weight, ept12, pad12, (NS,))
    ew12 = jnp.stack([ewr, ewr])
    zeros = jnp.zeros((RPT, 128), jnp.float32)

    agg_half = _make_edge_agg(pad0 // CH)
    agg_full = _make_edge_agg(pad12 // CH)

    x2 = jnp.concatenate([x, x], axis=0)
    agg0 = agg_half(x2, src0, dst0, ew0, zeros)
    wr0 = jnp.concatenate([W_rel0, W_rel0], axis=0)
    wo0 = jnp.concatenate([W_root0, jnp.zeros_like(W_root0)], axis=0)
    xs = jnp.concatenate([x, x], axis=0)
    y0, st0 = _y_call(agg0, xs, wr0, b_rel0.reshape(1, HID), wo0)
    h1 = _bn_call(y0, st0, gamma0.reshape(1, HID),
                  beta0.reshape(1, HID)).reshape(2 * N, HID // 2)

    agg1 = agg_full(h1, src12, dst12, ew12, zeros)
    y1, st1 = _y_call(agg1, h1, W_rel1, b_rel1.reshape(1, HID), W_root1)
    h2 = _bn_call(y1, st1, gamma1.reshape(1, HID),
                  beta1.reshape(1, HID)).reshape(2 * N, HID // 2)

    agg2 = agg_full(h2, src12, dst12, ew12, zeros)
    sumpool, maxpool = _pool_call(agg2, h2, W_rel2, b_rel2.reshape(1, HID),
                                  W_root2, batch.reshape(N, 1))

    W2p = jnp.pad(W2, ((0, 0), (0, 128 - OUT_DIM)))
    b2p = jnp.pad(b2, (0, 128 - OUT_DIM)).reshape(1, 128)
    out = _head_call(sumpool, maxpool, batch.reshape(N, 1), W1,
                     b1.reshape(1, HID), W2p, b2p)
    return out[:, :OUT_DIM]
